# baseline (speedup 1.0000x reference)
import functools

import jax
import jax.numpy as jnp
from jax import lax
from jax.experimental import pallas as pl
from jax.experimental.pallas import tpu as pltpu
from jax.experimental.pallas import tpu_sc as plsc

N = 10000
NP = 10240
E = 320000
CH = 128
NW = 32
NC = E // CH
CPW = 78
RPT = NP // 16
NB = 2
NGRP = CPW // NB
NBD = 4
NGRPD = CPW // NBD

_mesh = plsc.VectorSubcoreMesh(core_axis_name="c", subcore_axis_name="s")


def _fill(ref, value):
    v = jnp.full((16,), value, jnp.float32)

    def body(r, carry):
        for j in range(8):
            ref[r, pl.ds(j * 16, 16)] = v
        return carry

    lax.fori_loop(0, CH, body, 0)


@functools.partial(
    pl.kernel,
    out_type=jax.ShapeDtypeStruct((2 * NP, 128), jnp.float32),
    mesh=_mesh,
    scratch_types=[
        pltpu.VMEM((CH, 128), jnp.float32),
        [pltpu.VMEM((CH,), jnp.int32) for _ in range(NBD)],
        [pltpu.SemaphoreType.DMA for _ in range(NBD)],
        pltpu.VMEM_SHARED((NP, 128), jnp.float32),
    ],
)
def _sc_degree(dst_hbm, out_hbm, ones_v, dst2, sem_d, acc_sh):
    cid = lax.axis_index("c")
    sid = lax.axis_index("s")
    wid = sid * 2 + cid
    ebase = (wid * CPW + jnp.minimum(wid, 4)) * CH
    _fill(ones_v, 0.0)
    for j in range(RPT // CH):
        pltpu.sync_copy(ones_v, acc_sh.at[pl.ds(sid * RPT + j * CH, CH)])
    _fill(ones_v, 1.0)
    plsc.subcore_barrier()

    for b in range(NBD):
        pltpu.async_copy(dst_hbm.at[pl.ds(ebase + b * CH, CH)], dst2[b], sem_d[b])

    def outer(g, carry):
        for b in range(NBD):
            t = g * NBD + b
            pltpu.make_async_copy(dst_hbm.at[pl.ds(0, CH)], dst2[b], sem_d[b]).wait()
            pltpu.sync_copy(ones_v, acc_sh.at[dst2[b]], add=True)

            @pl.when(g < NGRPD - 1)
            def _():
                pltpu.async_copy(
                    dst_hbm.at[pl.ds(ebase + (t + NBD) * CH, CH)], dst2[b], sem_d[b])
        return carry

    lax.fori_loop(0, NGRPD, outer, 0)

    for t in (NGRPD * NBD, NGRPD * NBD + 1):
        pltpu.sync_copy(dst_hbm.at[pl.ds(ebase + t * CH, CH)], dst2[0])
        pltpu.sync_copy(ones_v, acc_sh.at[dst2[0]], add=True)

    @pl.when(wid < NC - NW * CPW)
    def _():
        pltpu.sync_copy(dst_hbm.at[pl.ds(ebase + CPW * CH, CH)], dst2[0])
        pltpu.sync_copy(ones_v, acc_sh.at[dst2[0]], add=True)

    plsc.subcore_barrier()
    pltpu.sync_copy(
        acc_sh.at[pl.ds(sid * RPT, RPT)],
        out_hbm.at[pl.ds(cid * NP + sid * RPT, RPT)],
    )


@functools.partial(
    pl.kernel,
    out_type=jax.ShapeDtypeStruct((2 * NP, 128), jnp.float32),
    mesh=_mesh,
    scratch_types=[
        pltpu.VMEM((CPW * CH,), jnp.int32),
        [pltpu.VMEM((CH,), jnp.int32) for _ in range(NB)],
        [pltpu.VMEM((CH, 128), jnp.float32) for _ in range(NB)],
        [pltpu.SemaphoreType.DMA for _ in range(NB)],
        pltpu.SemaphoreType.DMA,
        pltpu.VMEM_SHARED((NP, 128), jnp.float32),
    ],
)
def _sc_scatter(z_hbm, src_hbm, dst_hbm, out_hbm,
                src_all, dst2, rows2, sem_d, sem_g, acc_sh):
    cid = lax.axis_index("c")
    sid = lax.axis_index("s")
    wid = sid * 2 + cid
    ebase = (wid * CPW + jnp.minimum(wid, 4)) * CH
    _fill(rows2[0], 0.0)
    for j in range(RPT // CH):
        pltpu.sync_copy(rows2[0], acc_sh.at[pl.ds(sid * RPT + j * CH, CH)])
    pltpu.sync_copy(src_hbm.at[pl.ds(ebase, CPW * CH)], src_all)
    plsc.subcore_barrier()

    for b in range(NB):
        pltpu.async_copy(dst_hbm.at[pl.ds(ebase + b * CH, CH)], dst2[b], sem_d[b])
        pltpu.async_copy(z_hbm.at[src_all.at[pl.ds(b * CH, CH)]], rows2[b], sem_g)

    def outer(g, carry):
        for b in range(NB):
            t = g * NB + b
            pltpu.make_async_copy(dst_hbm.at[pl.ds(0, CH)], dst2[b], sem_d[b]).wait()
            pltpu.make_async_copy(z_hbm.at[pl.ds(0, CH)], rows2[b], sem_g).wait()
            pltpu.sync_copy(rows2[b], acc_sh.at[dst2[b]], add=True)

            @pl.when(g < NGRP - 1)
            def _():
                pltpu.async_copy(
                    dst_hbm.at[pl.ds(ebase + (t + NB) * CH, CH)], dst2[b], sem_d[b])
                pltpu.async_copy(
                    z_hbm.at[src_all.at[pl.ds((t + NB) * CH, CH)]], rows2[b], sem_g)
        return carry

    lax.fori_loop(0, NGRP, outer, 0)

    @pl.when(wid < NC - NW * CPW)
    def _():
        pltpu.sync_copy(dst_hbm.at[pl.ds(ebase + CPW * CH, CH)], dst2[0])
        pltpu.sync_copy(src_hbm.at[pl.ds(ebase + CPW * CH, CH)], dst2[1])
        pltpu.async_copy(z_hbm.at[dst2[1]], rows2[0], sem_g).wait()
        pltpu.sync_copy(rows2[0], acc_sh.at[dst2[0]], add=True)

    plsc.subcore_barrier()
    pltpu.sync_copy(
        acc_sh.at[pl.ds(sid * RPT, RPT)],
        out_hbm.at[pl.ds(cid * NP + sid * RPT, RPT)],
    )


BR = 256
OFF = NP // BR


def _tc_a_body(x_ref, da_ref, db_ref, o_ref, dis_ref):
    deg = da_ref[:, 0:1] + db_ref[:, 0:1] + 1.0
    dis = lax.rsqrt(deg)
    o_ref[...] = x_ref[...] * dis
    dis_ref[...] = dis[:, 0]


def _tc_b_body(aa_ref, ab_ref, z_ref, dis_ref, b1_ref, w1_ref, w2_ref, o_ref):
    i = pl.program_id(0)
    dis = dis_ref[...][:, None]
    u = (aa_ref[...] + ab_ref[...] + z_ref[...]) * dis
    h = jnp.dot(u, w1_ref[...], preferred_element_type=jnp.float32,
                precision=lax.Precision.HIGHEST) + b1_ref[...]
    h = jnp.maximum(h, 0.0)
    row = lax.broadcasted_iota(jnp.int32, (BR, 1), 0) + i * BR
    h = jnp.where(row < N, h, 0.0)
    z2 = jnp.dot(h, w2_ref[...], preferred_element_type=jnp.float32,
                 precision=lax.Precision.HIGHEST)
    o_ref[...] = z2 * dis


def _tc_c_body(aa_ref, ab_ref, z_ref, dis_ref, b_ref, o_ref):
    dis = dis_ref[...][:, None]
    o_ref[...] = (aa_ref[...] + ab_ref[...] + z_ref[...]) * dis + b_ref[...]


def _tc_a(x_p, deg16):
    return pl.pallas_call(
        _tc_a_body,
        grid=(NP // BR,),
        in_specs=[
            pl.BlockSpec((BR, 128), lambda i: (i, 0)),
            pl.BlockSpec((BR, 128), lambda i: (i, 0)),
            pl.BlockSpec((BR, 128), lambda i: (i + OFF, 0)),
        ],
        out_specs=[
            pl.BlockSpec((BR, 128), lambda i: (i, 0)),
            pl.BlockSpec((BR,), lambda i: (i,)),
        ],
        out_shape=[
            jax.ShapeDtypeStruct((NP, 128), jnp.float32),
            jax.ShapeDtypeStruct((NP,), jnp.float32),
        ],
    )(x_p, deg16, deg16)


def _tc_b(acc2, z1, dis, b1, W1, W2):
    return pl.pallas_call(
        _tc_b_body,
        grid=(NP // BR,),
        in_specs=[
            pl.BlockSpec((BR, 128), lambda i: (i, 0)),
            pl.BlockSpec((BR, 128), lambda i: (i + OFF, 0)),
            pl.BlockSpec((BR, 128), lambda i: (i, 0)),
            pl.BlockSpec((BR,), lambda i: (i,)),
            pl.BlockSpec((128,), lambda i: (0,)),
            pl.BlockSpec((128, 128), lambda i: (0, 0)),
            pl.BlockSpec((128, 128), lambda i: (0, 0)),
        ],
        out_specs=pl.BlockSpec((BR, 128), lambda i: (i, 0)),
        out_shape=jax.ShapeDtypeStruct((NP, 128), jnp.float32),
    )(acc2, acc2, z1, dis, b1, W1, W2)


def _tc_c(acc2, z2, dis, b2):
    return pl.pallas_call(
        _tc_c_body,
        grid=(NP // BR,),
        in_specs=[
            pl.BlockSpec((BR, 128), lambda i: (i, 0)),
            pl.BlockSpec((BR, 128), lambda i: (i + OFF, 0)),
            pl.BlockSpec((BR, 128), lambda i: (i, 0)),
            pl.BlockSpec((BR,), lambda i: (i,)),
            pl.BlockSpec((128,), lambda i: (0,)),
        ],
        out_specs=pl.BlockSpec((BR, 128), lambda i: (i, 0)),
        out_shape=jax.ShapeDtypeStruct((NP, 128), jnp.float32),
    )(acc2, acc2, z2, dis, b2)


def kernel(x, edge_index, W1, b1, W2, b2):
    src = edge_index[0].astype(jnp.int32)
    dst = edge_index[1].astype(jnp.int32)
    x_p = jnp.zeros((NP, 128), jnp.float32).at[:N].set(x)

    deg16 = _sc_degree(dst)
    zx, dis = _tc_a(x_p, deg16)
    acc1 = _sc_scatter(zx, src, dst)
    z2 = _tc_b(acc1, zx, dis, b1, W1, W2)
    acc2 = _sc_scatter(z2, src, dst)
    return _tc_c(acc2, z2, dis, b2)[:N]


# speedup vs baseline: 1.0142x; 1.0142x over previous
import functools

import jax
import jax.numpy as jnp
from jax import lax
from jax.experimental import pallas as pl
from jax.experimental.pallas import tpu as pltpu
from jax.experimental.pallas import tpu_sc as plsc

N = 10000
NP = 10240
E = 320000
CH = 128
NW = 32
NC = E // CH
CPW = 78
RPT = NP // 16
NB = 2
NGRP = CPW // NB

_mesh = plsc.VectorSubcoreMesh(core_axis_name="c", subcore_axis_name="s")


def _fill(ref, value):
    v = jnp.full((16,), value, jnp.float32)

    def body(r, carry):
        for j in range(8):
            ref[r, pl.ds(j * 16, 16)] = v
        return carry

    lax.fori_loop(0, CH, body, 0)


@functools.partial(
    pl.kernel,
    out_type=jax.ShapeDtypeStruct((2 * NP, 128), jnp.float32),
    mesh=_mesh,
    scratch_types=[
        pltpu.VMEM((CH, 128), jnp.float32),
        [pltpu.VMEM((CH,), jnp.int32) for _ in range(NB)],
        [pltpu.SemaphoreType.DMA for _ in range(NB)],
        pltpu.VMEM_SHARED((NP, 128), jnp.float32),
    ],
)
def _sc_degree(dst_hbm, out_hbm, ones_v, dst2, sem_d, acc_sh):
    cid = lax.axis_index("c")
    sid = lax.axis_index("s")
    wid = sid * 2 + cid
    ebase = (wid * CPW + jnp.minimum(wid, 4)) * CH
    _fill(ones_v, 0.0)
    for j in range(RPT // CH):
        pltpu.sync_copy(ones_v, acc_sh.at[pl.ds(sid * RPT + j * CH, CH)])
    _fill(ones_v, 1.0)
    plsc.subcore_barrier()

    for b in range(NB):
        pltpu.async_copy(dst_hbm.at[pl.ds(ebase + b * CH, CH)], dst2[b], sem_d[b])

    def outer(g, carry):
        for b in range(NB):
            t = g * NB + b
            pltpu.make_async_copy(dst_hbm.at[pl.ds(0, CH)], dst2[b], sem_d[b]).wait()
            pltpu.sync_copy(ones_v, acc_sh.at[dst2[b]], add=True)

            @pl.when(g < NGRP - 1)
            def _():
                pltpu.async_copy(
                    dst_hbm.at[pl.ds(ebase + (t + NB) * CH, CH)], dst2[b], sem_d[b])
        return carry

    lax.fori_loop(0, NGRP, outer, 0)

    @pl.when(wid < NC - NW * CPW)
    def _():
        pltpu.sync_copy(dst_hbm.at[pl.ds(ebase + CPW * CH, CH)], dst2[0])
        pltpu.sync_copy(ones_v, acc_sh.at[dst2[0]], add=True)

    plsc.subcore_barrier()
    pltpu.sync_copy(
        acc_sh.at[pl.ds(sid * RPT, RPT)],
        out_hbm.at[pl.ds(cid * NP + sid * RPT, RPT)],
    )


@functools.partial(
    pl.kernel,
    out_type=jax.ShapeDtypeStruct((2 * NP, 128), jnp.float32),
    mesh=_mesh,
    scratch_types=[
        pltpu.VMEM((CPW * CH,), jnp.int32),
        [pltpu.VMEM((CH,), jnp.int32) for _ in range(NB)],
        [pltpu.VMEM((CH, 128), jnp.float32) for _ in range(NB)],
        [pltpu.SemaphoreType.DMA for _ in range(NB)],
        pltpu.SemaphoreType.DMA,
        pltpu.VMEM_SHARED((NP, 128), jnp.float32),
    ],
)
def _sc_scatter(z_hbm, src_hbm, dst_hbm, out_hbm,
                src_all, dst2, rows2, sem_d, sem_g, acc_sh):
    cid = lax.axis_index("c")
    sid = lax.axis_index("s")
    wid = sid * 2 + cid
    ebase = (wid * CPW + jnp.minimum(wid, 4)) * CH
    _fill(rows2[0], 0.0)
    for j in range(RPT // CH):
        pltpu.sync_copy(rows2[0], acc_sh.at[pl.ds(sid * RPT + j * CH, CH)])
    pltpu.sync_copy(src_hbm.at[pl.ds(ebase, CPW * CH)], src_all)
    plsc.subcore_barrier()

    for b in range(NB):
        pltpu.async_copy(dst_hbm.at[pl.ds(ebase + b * CH, CH)], dst2[b], sem_d[b])
        pltpu.async_copy(z_hbm.at[src_all.at[pl.ds(b * CH, CH)]], rows2[b], sem_g)

    def outer(g, carry):
        for b in range(NB):
            t = g * NB + b
            pltpu.make_async_copy(dst_hbm.at[pl.ds(0, CH)], dst2[b], sem_d[b]).wait()
            pltpu.make_async_copy(z_hbm.at[pl.ds(0, CH)], rows2[b], sem_g).wait()
            pltpu.sync_copy(rows2[b], acc_sh.at[dst2[b]], add=True)

            @pl.when(g < NGRP - 1)
            def _():
                pltpu.async_copy(
                    dst_hbm.at[pl.ds(ebase + (t + NB) * CH, CH)], dst2[b], sem_d[b])
                pltpu.async_copy(
                    z_hbm.at[src_all.at[pl.ds((t + NB) * CH, CH)]], rows2[b], sem_g)
        return carry

    lax.fori_loop(0, NGRP, outer, 0)

    @pl.when(wid < NC - NW * CPW)
    def _():
        pltpu.sync_copy(dst_hbm.at[pl.ds(ebase + CPW * CH, CH)], dst2[0])
        pltpu.sync_copy(src_hbm.at[pl.ds(ebase + CPW * CH, CH)], dst2[1])
        pltpu.async_copy(z_hbm.at[dst2[1]], rows2[0], sem_g).wait()
        pltpu.sync_copy(rows2[0], acc_sh.at[dst2[0]], add=True)

    plsc.subcore_barrier()
    pltpu.sync_copy(
        acc_sh.at[pl.ds(sid * RPT, RPT)],
        out_hbm.at[pl.ds(cid * NP + sid * RPT, RPT)],
    )


BR = 256
OFF = NP // BR


def _tc_a_body(x_ref, w_ref, da_ref, db_ref, o_ref, dis_ref):
    deg = da_ref[:, 0:1] + db_ref[:, 0:1] + 1.0
    dis = lax.rsqrt(deg)
    xw = jnp.dot(x_ref[...], w_ref[...], preferred_element_type=jnp.float32,
                 precision=lax.Precision.HIGHEST)
    o_ref[...] = xw * dis
    dis_ref[...] = dis[:, 0]


def _tc_b_body(aa_ref, ab_ref, z_ref, dis_ref, b_ref, w_ref, o_ref):
    i = pl.program_id(0)
    dis = dis_ref[...][:, None]
    s = (aa_ref[...] + ab_ref[...] + z_ref[...]) * dis + b_ref[...]
    h = jnp.maximum(s, 0.0)
    row = lax.broadcasted_iota(jnp.int32, (BR, 1), 0) + i * BR
    h = jnp.where(row < N, h, 0.0)
    z2 = jnp.dot(h, w_ref[...], preferred_element_type=jnp.float32,
                 precision=lax.Precision.HIGHEST)
    o_ref[...] = z2 * dis


def _tc_c_body(aa_ref, ab_ref, z_ref, dis_ref, b_ref, o_ref):
    dis = dis_ref[...][:, None]
    o_ref[...] = (aa_ref[...] + ab_ref[...] + z_ref[...]) * dis + b_ref[...]


def _tc_a(x_p, W, deg16):
    return pl.pallas_call(
        _tc_a_body,
        grid=(NP // BR,),
        in_specs=[
            pl.BlockSpec((BR, 128), lambda i: (i, 0)),
            pl.BlockSpec((128, 128), lambda i: (0, 0)),
            pl.BlockSpec((BR, 128), lambda i: (i, 0)),
            pl.BlockSpec((BR, 128), lambda i: (i + OFF, 0)),
        ],
        out_specs=[
            pl.BlockSpec((BR, 128), lambda i: (i, 0)),
            pl.BlockSpec((BR,), lambda i: (i,)),
        ],
        out_shape=[
            jax.ShapeDtypeStruct((NP, 128), jnp.float32),
            jax.ShapeDtypeStruct((NP,), jnp.float32),
        ],
    )(x_p, W, deg16, deg16)


def _tc_b(acc2, z1, dis, b1, W2):
    return pl.pallas_call(
        _tc_b_body,
        grid=(NP // BR,),
        in_specs=[
            pl.BlockSpec((BR, 128), lambda i: (i, 0)),
            pl.BlockSpec((BR, 128), lambda i: (i + OFF, 0)),
            pl.BlockSpec((BR, 128), lambda i: (i, 0)),
            pl.BlockSpec((BR,), lambda i: (i,)),
            pl.BlockSpec((128,), lambda i: (0,)),
            pl.BlockSpec((128, 128), lambda i: (0, 0)),
        ],
        out_specs=pl.BlockSpec((BR, 128), lambda i: (i, 0)),
        out_shape=jax.ShapeDtypeStruct((NP, 128), jnp.float32),
    )(acc2, acc2, z1, dis, b1, W2)


def _tc_c(acc2, z2, dis, b2):
    return pl.pallas_call(
        _tc_c_body,
        grid=(NP // BR,),
        in_specs=[
            pl.BlockSpec((BR, 128), lambda i: (i, 0)),
            pl.BlockSpec((BR, 128), lambda i: (i + OFF, 0)),
            pl.BlockSpec((BR, 128), lambda i: (i, 0)),
            pl.BlockSpec((BR,), lambda i: (i,)),
            pl.BlockSpec((128,), lambda i: (0,)),
        ],
        out_specs=pl.BlockSpec((BR, 128), lambda i: (i, 0)),
        out_shape=jax.ShapeDtypeStruct((NP, 128), jnp.float32),
    )(acc2, acc2, z2, dis, b2)


def kernel(x, edge_index, W1, b1, W2, b2):
    src = edge_index[0].astype(jnp.int32)
    dst = edge_index[1].astype(jnp.int32)
    x_p = jnp.zeros((NP, 128), jnp.float32).at[:N].set(x)

    deg16 = _sc_degree(dst)
    z1, dis = _tc_a(x_p, W1, deg16)
    acc1 = _sc_scatter(z1, src, dst)
    z2 = _tc_b(acc1, z1, dis, b1, W2)
    acc2 = _sc_scatter(z2, src, dst)
    return _tc_c(acc2, z2, dis, b2)[:N]
